# baseline (speedup 1.0000x reference)
import functools

import jax
import jax.numpy as jnp
from jax import lax
from jax.experimental import pallas as pl
from jax.experimental.pallas import tpu as pltpu
from jax.experimental.pallas import tpu_sc as plsc

EMB_DIM = 64
NC, NS = 2, 16
NW = NC * NS
CHUNK = 128
NBUF = 4


@functools.lru_cache(maxsize=None)
def _build_gather(n_chunks: int):
    b_per_w = n_chunks * CHUNK
    n_rows = NW * b_per_w
    mesh = plsc.VectorSubcoreMesh(core_axis_name="c", subcore_axis_name="s")

    def body(idx_hbm, table_hbm, out_hbm, idx_v, r0, r1, r2, r3,
             s0, s1, s2, s3):
        rows = (r0, r1, r2, r3)
        sems = (s0, s1, s2, s3)
        wid = lax.axis_index("s") * NC + lax.axis_index("c")
        base = wid * b_per_w

        pltpu.sync_copy(idx_hbm.at[wid], idx_v)

        for b in range(NBUF):
            pltpu.async_copy(table_hbm.at[idx_v.at[b]], rows[b], sems[b])

        @pl.loop(0, n_chunks, step=NBUF)
        def _(c0):
            for b in range(NBUF):
                c = c0 + b
                pltpu.make_async_copy(
                    table_hbm.at[idx_v.at[c]], rows[b], sems[b]).wait()
                pltpu.sync_copy(
                    rows[b], out_hbm.at[pl.ds(base + c * CHUNK, CHUNK)])
                nxt = c + NBUF

                @pl.when(nxt < n_chunks)
                def _():
                    pltpu.async_copy(
                        table_hbm.at[idx_v.at[nxt]], rows[b], sems[b])

    return pl.kernel(
        body,
        mesh=mesh,
        compiler_params=pltpu.CompilerParams(use_tc_tiling_on_sc=False),
        out_type=jax.ShapeDtypeStruct((n_rows, EMB_DIM), jnp.float32),
        scratch_types=[
            pltpu.VMEM((n_chunks, CHUNK), jnp.int32),
            pltpu.VMEM((CHUNK, EMB_DIM), jnp.float32),
            pltpu.VMEM((CHUNK, EMB_DIM), jnp.float32),
            pltpu.VMEM((CHUNK, EMB_DIM), jnp.float32),
            pltpu.VMEM((CHUNK, EMB_DIM), jnp.float32),
            pltpu.SemaphoreType.DMA,
            pltpu.SemaphoreType.DMA,
            pltpu.SemaphoreType.DMA,
            pltpu.SemaphoreType.DMA,
        ],
    )


def kernel(token_ids, weight):
    orig_shape = token_ids.shape
    flat = token_ids.reshape(-1).astype(jnp.int32)
    n = flat.shape[0]
    tile = NW * CHUNK
    n_pad = -(-n // tile) * tile
    if n_pad != n:
        flat = jnp.pad(flat, (0, n_pad - n))
    n_chunks = n_pad // tile
    idx3 = flat.reshape(NW, n_chunks, CHUNK)
    out = _build_gather(n_chunks)(idx3, weight)
    if n_pad != n:
        out = out[:n]
    return out.reshape(*orig_shape, EMB_DIM)
